# baseline (speedup 1.0000x reference)
import functools
import math

import jax
import jax.numpy as jnp
from jax import lax
from jax.experimental import pallas as pl
from jax.experimental.pallas import tpu as pltpu
from jax.experimental.pallas import tpu_sc as plsc

_LANES = 16


def _make_sc_kernel(N, D, NW, NC, S, G, QL, NGRP):
    scale = math.sqrt(D)
    nj = D // _LANES
    nh = S // QL
    rows_per_tile = NGRP * G * QL
    mesh = plsc.VectorSubcoreMesh(core_axis_name="c", subcore_axis_name="s")

    @functools.partial(
        pl.kernel,
        mesh=mesh,
        out_type=jax.ShapeDtypeStruct((N // S, nh, QL, D), jnp.float32),
        scratch_types=[
            pltpu.VMEM((rows_per_tile,), jnp.int32),
            pltpu.VMEM((2, G, 1, QL, D), jnp.float32),
            pltpu.VMEM((S, D), jnp.float32),
            pltpu.SemaphoreType.DMA,
            pltpu.SemaphoreType.DMA,
            pltpu.SemaphoreType.DMA,
            pltpu.SemaphoreType.DMA,
        ],
    )
    def k(w_hbm, idx_hbm, pe_hbm, out_hbm, idx_v, rows_v, pe_v, g0, g1, s0, s1):
        wid = lax.axis_index("s") * NC + lax.axis_index("c")
        base = wid * rows_per_tile
        gsems = (g0, g1)
        ssems = (s0, s1)

        pltpu.sync_copy(pe_hbm, pe_v)
        pltpu.sync_copy(idx_hbm.at[pl.ds(base, rows_per_tile)], idx_v)

        def loc0_of(g3, gg):
            q = g3 // nh
            h = lax.rem(g3, nh)
            return q * (G * S) + gg * S + h * QL

        def start_gathers(g3, b):
            for gg in range(G):
                pltpu.async_copy(
                    w_hbm.at[idx_v.at[pl.ds(loc0_of(g3, gg), QL)]],
                    rows_v.at[b, gg, 0],
                    gsems[b],
                )

        def wait_gathers(g3, b):
            for gg in range(G):
                pltpu.make_async_copy(
                    w_hbm.at[idx_v.at[pl.ds(loc0_of(g3, gg), QL)]],
                    rows_v.at[b, gg, 0],
                    gsems[b],
                ).wait()

        def out_slice(g3):
            q = g3 // nh
            h = lax.rem(g3, nh)
            seq0 = wid * (rows_per_tile // S) + q * G
            return out_hbm.at[pl.ds(seq0, G), pl.ds(h, 1)]

        def start_stores(g3, b):
            pltpu.async_copy(rows_v.at[b], out_slice(g3), ssems[b])

        def wait_stores(g3, b):
            pltpu.make_async_copy(rows_v.at[b], out_slice(g3), ssems[b]).wait()

        start_gathers(0, 0)

        def group_body(g3, b):
            nb = 1 - b

            @pl.when(g3 >= 1)
            def _():
                wait_stores(g3 - 1, nb)

            @pl.when(g3 + 1 < NGRP)
            def _():
                start_gathers(g3 + 1, nb)

            wait_gathers(g3, b)

            p0 = lax.rem(g3, nh) * QL

            @plsc.parallel_loop(0, QL, unroll=2)
            def _(i):
                for j in range(nj):
                    sl = pl.ds(j * _LANES, _LANES)
                    p = pe_v[p0 + i, sl]
                    for gg in range(G):
                        r = rows_v[b, gg, 0, i, sl]
                        rows_v[b, gg, 0, i, sl] = r * scale + p

            start_stores(g3, b)

        def outer(t, carry):
            for b in range(2):
                group_body(t * 2 + b, b)
            return carry

        lax.fori_loop(0, NGRP // 2, outer, 0)

        wait_stores(NGRP - 1, 1)

    return k


def kernel(input_seq, W, pos_emb):
    B, S = input_seq.shape
    _, D = W.shape
    N = B * S

    info = plsc.get_sparse_core_info()
    NC, NS = info.num_cores, info.num_subcores
    NW = NC * NS

    G = 8
    QL = 40
    assert D % _LANES == 0
    assert S % QL == 0 and QL % 8 == 0
    assert B % (NW * G) == 0
    NGRP = (B // NW // G) * (S // QL)
    assert NGRP % 2 == 0

    idx = input_seq.astype(jnp.int32).reshape(-1)
    pe = pos_emb[:S].astype(jnp.float32)

    k = _make_sc_kernel(N, D, NW, NC, S, G, QL, NGRP)
    out = k(W, idx, pe)
    return out.reshape(B, S, D)


# speedup vs baseline: 1.0050x; 1.0050x over previous
import functools
import math

import jax
import jax.numpy as jnp
from jax import lax
from jax.experimental import pallas as pl
from jax.experimental.pallas import tpu as pltpu
from jax.experimental.pallas import tpu_sc as plsc

_LANES = 16


def _make_sc_kernel(N, D, NW, NC, S, G, QL, NGRP):
    scale = math.sqrt(D)
    nj = D // _LANES
    nh = S // QL
    rows_per_tile = NGRP * G * QL
    mesh = plsc.VectorSubcoreMesh(core_axis_name="c", subcore_axis_name="s")

    @functools.partial(
        pl.kernel,
        mesh=mesh,
        out_type=jax.ShapeDtypeStruct((N // S, nh, QL, D), jnp.float32),
        scratch_types=[
            pltpu.VMEM((rows_per_tile,), jnp.int32),
            pltpu.VMEM((2, G, 1, QL, D), jnp.float32),
            pltpu.VMEM((S, D), jnp.float32),
            pltpu.SemaphoreType.DMA,
            pltpu.SemaphoreType.DMA,
            pltpu.SemaphoreType.DMA,
            pltpu.SemaphoreType.DMA,
        ],
    )
    def k(w_hbm, idx_hbm, pe_hbm, out_hbm, idx_v, rows_v, pe_v, g0, g1, s0, s1):
        wid = lax.axis_index("s") * NC + lax.axis_index("c")
        seq_per_tile = rows_per_tile // S
        gsems = (g0, g1)
        ssems = (s0, s1)

        pltpu.sync_copy(pe_hbm.at[pl.ds(0, S)], pe_v)
        pltpu.sync_copy(idx_hbm.at[pl.ds(wid * rows_per_tile, rows_per_tile)],
                        idx_v)

        def idx_slice(g3, gg):
            q = g3 // nh
            h = lax.rem(g3, nh)
            return idx_v.at[pl.ds(q * (G * S) + gg * S + h * QL, QL)]

        def start_gathers(g3, b):
            for gg in range(G):
                pltpu.async_copy(
                    w_hbm.at[idx_slice(g3, gg)],
                    rows_v.at[b, gg, 0],
                    gsems[b],
                )

        def wait_gathers(g3, b):
            for gg in range(G):
                pltpu.make_async_copy(
                    w_hbm.at[idx_slice(g3, gg)],
                    rows_v.at[b, gg, 0],
                    gsems[b],
                ).wait()

        def out_slice(g3):
            q = g3 // nh
            h = lax.rem(g3, nh)
            seq0 = wid * (rows_per_tile // S) + q * G
            return out_hbm.at[pl.ds(seq0, G), pl.ds(h, 1)]

        def start_stores(g3, b):
            pltpu.async_copy(rows_v.at[b], out_slice(g3), ssems[b])

        def wait_stores(g3, b):
            pltpu.make_async_copy(rows_v.at[b], out_slice(g3), ssems[b]).wait()

        start_gathers(0, 0)

        def group_body(g3, b):
            nb = 1 - b

            @pl.when(g3 >= 1)
            def _():
                wait_stores(g3 - 1, nb)

            @pl.when(g3 + 1 < NGRP)
            def _():
                start_gathers(g3 + 1, nb)

            wait_gathers(g3, b)

            p0 = lax.rem(g3, nh) * QL

            @plsc.parallel_loop(0, QL, unroll=2)
            def _(i):
                for j in range(nj):
                    sl = pl.ds(j * _LANES, _LANES)
                    p = pe_v[p0 + i, sl]
                    for gg in range(G):
                        r = rows_v[b, gg, 0, i, sl]
                        rows_v[b, gg, 0, i, sl] = r * scale + p

            start_stores(g3, b)

        def outer(t, carry):
            for b in range(2):
                group_body(t * 2 + b, b)
            return carry

        lax.fori_loop(0, NGRP // 2, outer, 0)

        wait_stores(NGRP - 1, 1)

    return k


def kernel(input_seq, W, pos_emb):
    B, S = input_seq.shape
    _, D = W.shape
    N = B * S

    info = plsc.get_sparse_core_info()
    NC, NS = info.num_cores, info.num_subcores
    NW = NC * NS

    G = 8
    QL = 40
    assert D % _LANES == 0
    assert S % QL == 0 and QL % 8 == 0
    assert B % (NW * G) == 0
    NGRP = (B // NW // G) * (S // QL)
    assert NGRP % 2 == 0

    k = _make_sc_kernel(N, D, NW, NC, S, G, QL, NGRP)
    idx = input_seq.astype(jnp.int32).reshape(-1)
    out = k(W, idx, pos_emb.astype(jnp.float32))
    return out.reshape(B, S, D)


# speedup vs baseline: 1.0078x; 1.0028x over previous
import functools
import math

import jax
import jax.numpy as jnp
from jax import lax
from jax.experimental import pallas as pl
from jax.experimental.pallas import tpu as pltpu
from jax.experimental.pallas import tpu_sc as plsc

_LANES = 16


def _make_sc_kernel(N, D, NW, NC, S, G, QL, NGRP):
    scale = math.sqrt(D)
    nj = D // _LANES
    nh = S // QL
    rows_per_tile = NGRP * G * QL
    mesh = plsc.VectorSubcoreMesh(core_axis_name="c", subcore_axis_name="s")

    @functools.partial(
        pl.kernel,
        mesh=mesh,
        out_type=jax.ShapeDtypeStruct((N // S, S, D), jnp.float32),
        scratch_types=[
            pltpu.VMEM((rows_per_tile,), jnp.int32),
            pltpu.VMEM((2, G, QL, D), jnp.float32),
            pltpu.VMEM((S, D), jnp.float32),
            pltpu.SemaphoreType.DMA,
            pltpu.SemaphoreType.DMA,
            pltpu.SemaphoreType.DMA,
            pltpu.SemaphoreType.DMA,
        ],
    )
    def k(w_hbm, idx_hbm, pe_hbm, out_hbm, idx_v, rows_v, pe_v, g0, g1, s0, s1):
        wid = lax.axis_index("s") * NC + lax.axis_index("c")
        seq_per_tile = rows_per_tile // S
        gsems = (g0, g1)
        ssems = (s0, s1)

        pltpu.sync_copy(pe_hbm.at[pl.ds(0, S)], pe_v)
        pltpu.sync_copy(idx_hbm.at[pl.ds(wid * rows_per_tile, rows_per_tile)],
                        idx_v)

        def idx_slice(g3, gg):
            q = g3 // nh
            h = lax.rem(g3, nh)
            return idx_v.at[pl.ds(q * (G * S) + gg * S + h * QL, QL)]

        def start_gathers(g3, b):
            for gg in range(G):
                pltpu.async_copy(
                    w_hbm.at[idx_slice(g3, gg)],
                    rows_v.at[b, gg],
                    gsems[b],
                )

        def wait_gathers(g3, b):
            for gg in range(G):
                pltpu.make_async_copy(
                    w_hbm.at[idx_slice(g3, gg)],
                    rows_v.at[b, gg],
                    gsems[b],
                ).wait()

        def out_slice(g3):
            q = g3 // nh
            h = lax.rem(g3, nh)
            seq0 = wid * (rows_per_tile // S) + q * G
            return out_hbm.at[pl.ds(seq0, G), pl.ds(h * QL, QL)]

        def start_stores(g3, b):
            pltpu.async_copy(rows_v.at[b], out_slice(g3), ssems[b])

        def wait_stores(g3, b):
            pltpu.make_async_copy(rows_v.at[b], out_slice(g3), ssems[b]).wait()

        start_gathers(0, 0)

        def group_body(g3, b):
            nb = 1 - b

            @pl.when(g3 >= 1)
            def _():
                wait_stores(g3 - 1, nb)

            @pl.when(g3 + 1 < NGRP)
            def _():
                start_gathers(g3 + 1, nb)

            wait_gathers(g3, b)

            p0 = lax.rem(g3, nh) * QL

            @plsc.parallel_loop(0, QL, unroll=2)
            def _(i):
                for j in range(nj):
                    sl = pl.ds(j * _LANES, _LANES)
                    p = pe_v[p0 + i, sl]
                    for gg in range(G):
                        r = rows_v[b, gg, i, sl]
                        rows_v[b, gg, i, sl] = r * scale + p

            start_stores(g3, b)

        def outer(t, carry):
            for b in range(2):
                group_body(t * 2 + b, b)
            return carry

        lax.fori_loop(0, NGRP // 2, outer, 0)

        wait_stores(NGRP - 1, 1)

    return k


def kernel(input_seq, W, pos_emb):
    B, S = input_seq.shape
    _, D = W.shape
    N = B * S

    info = plsc.get_sparse_core_info()
    NC, NS = info.num_cores, info.num_subcores
    NW = NC * NS

    G = 8
    QL = 40
    assert D % _LANES == 0
    assert S % QL == 0 and QL % 8 == 0
    assert B % (NW * G) == 0
    NGRP = (B // NW // G) * (S // QL)
    assert NGRP % 2 == 0

    k = _make_sc_kernel(N, D, NW, NC, S, G, QL, NGRP)
    idx = input_seq.astype(jnp.int32).reshape(-1)
    return k(W, idx, pos_emb.astype(jnp.float32))


# speedup vs baseline: 1.0269x; 1.0189x over previous
import functools
import math

import jax
import jax.numpy as jnp
from jax import lax
from jax.experimental import pallas as pl
from jax.experimental.pallas import tpu as pltpu
from jax.experimental.pallas import tpu_sc as plsc

_LANES = 16


def _make_sc_kernel(N, D, NW, NC, S, G, QL, NGRP):
    scale = math.sqrt(D)
    nj = D // _LANES
    nh = S // QL
    rows_per_tile = NGRP * G * QL
    mesh = plsc.VectorSubcoreMesh(core_axis_name="c", subcore_axis_name="s")

    @functools.partial(
        pl.kernel,
        mesh=mesh,
        out_type=jax.ShapeDtypeStruct((N // S, S, D), jnp.float32),
        scratch_types=[
            pltpu.VMEM((rows_per_tile,), jnp.int32),
            pltpu.VMEM((2, G, QL, D), jnp.float32),
            pltpu.VMEM((S, D), jnp.float32),
            pltpu.SemaphoreType.DMA,
            pltpu.SemaphoreType.DMA,
            pltpu.SemaphoreType.DMA,
            pltpu.SemaphoreType.DMA,
        ],
    )
    def k(w_hbm, idx_hbm, pe_hbm, out_hbm, idx_v, rows_v, pe_v, g0, g1, s0, s1):
        wid = lax.axis_index("s") * NC + lax.axis_index("c")
        seq_per_tile = rows_per_tile // S
        gsems = (g0, g1)
        ssems = (s0, s1)

        pltpu.sync_copy(idx_hbm.at[pl.ds(wid * rows_per_tile, rows_per_tile)],
                        idx_v)

        def idx_slice(g3, gg):
            q = g3 // nh
            h = lax.rem(g3, nh)
            return idx_v.at[pl.ds(q * (G * S) + gg * S + h * QL, QL)]

        def start_gathers(g3, b):
            for gg in range(G):
                pltpu.async_copy(
                    w_hbm.at[idx_slice(g3, gg)],
                    rows_v.at[b, gg],
                    gsems[b],
                )

        def wait_gathers(g3, b):
            for gg in range(G):
                pltpu.make_async_copy(
                    w_hbm.at[idx_slice(g3, gg)],
                    rows_v.at[b, gg],
                    gsems[b],
                ).wait()

        def out_slice(g3):
            q = g3 // nh
            h = lax.rem(g3, nh)
            seq0 = wid * (rows_per_tile // S) + q * G
            return out_hbm.at[pl.ds(seq0, G), pl.ds(h * QL, QL)]

        def start_stores(g3, b):
            pltpu.async_copy(rows_v.at[b], out_slice(g3), ssems[b])

        def wait_stores(g3, b):
            pltpu.make_async_copy(rows_v.at[b], out_slice(g3), ssems[b]).wait()

        start_gathers(0, 0)
        pltpu.sync_copy(pe_hbm.at[pl.ds(0, S)], pe_v)

        def group_body(g3, b):
            nb = 1 - b

            @pl.when(g3 >= 1)
            def _():
                wait_stores(g3 - 1, nb)

            @pl.when(g3 + 1 < NGRP)
            def _():
                start_gathers(g3 + 1, nb)

            wait_gathers(g3, b)

            p0 = lax.rem(g3, nh) * QL

            @plsc.parallel_loop(0, QL, unroll=2)
            def _(i):
                for j in range(nj):
                    sl = pl.ds(j * _LANES, _LANES)
                    p = pe_v[p0 + i, sl]
                    for gg in range(G):
                        r = rows_v[b, gg, i, sl]
                        rows_v[b, gg, i, sl] = r * scale + p

            start_stores(g3, b)

        def outer(t, carry):
            for b in range(2):
                group_body(t * 2 + b, b)
            return carry

        lax.fori_loop(0, NGRP // 2, outer, 0)

        wait_stores(NGRP - 1, 1)

    return k


def kernel(input_seq, W, pos_emb):
    B, S = input_seq.shape
    _, D = W.shape
    N = B * S

    info = plsc.get_sparse_core_info()
    NC, NS = info.num_cores, info.num_subcores
    NW = NC * NS

    G = 8
    QL = 40
    assert D % _LANES == 0
    assert S % QL == 0 and QL % 8 == 0
    assert B % (NW * G) == 0
    NGRP = (B // NW // G) * (S // QL)
    assert NGRP % 2 == 0

    k = _make_sc_kernel(N, D, NW, NC, S, G, QL, NGRP)
    idx = input_seq.astype(jnp.int32).reshape(-1)
    return k(W, idx, pos_emb.astype(jnp.float32))


# speedup vs baseline: 1.0295x; 1.0025x over previous
import functools
import math

import jax
import jax.numpy as jnp
from jax import lax
from jax.experimental import pallas as pl
from jax.experimental.pallas import tpu as pltpu
from jax.experimental.pallas import tpu_sc as plsc

_LANES = 16


def _make_sc_kernel(N, D, NW, NC, S, G, QL, NGRP):
    scale = math.sqrt(D)
    nj = D // _LANES
    nh = S // QL
    rows_per_tile = NGRP * G * QL
    mesh = plsc.VectorSubcoreMesh(core_axis_name="c", subcore_axis_name="s")

    @functools.partial(
        pl.kernel,
        mesh=mesh,
        out_type=jax.ShapeDtypeStruct((N // S, S, D), jnp.float32),
        scratch_types=[
            pltpu.VMEM((rows_per_tile,), jnp.int32),
            pltpu.VMEM((2, G, QL, D), jnp.float32),
            pltpu.VMEM((S, D), jnp.float32),
            pltpu.SemaphoreType.DMA,
            pltpu.SemaphoreType.DMA,
            pltpu.SemaphoreType.DMA,
            pltpu.SemaphoreType.DMA,
        ],
    )
    def k(w_hbm, idx_hbm, pe_hbm, out_hbm, idx_v, rows_v, pe_v, g0, g1, s0, s1):
        wid = lax.axis_index("s") * NC + lax.axis_index("c")
        seq_per_tile = rows_per_tile // S
        gsems = (g0, g1)
        ssems = (s0, s1)

        pltpu.sync_copy(idx_hbm.at[pl.ds(wid * rows_per_tile, rows_per_tile)],
                        idx_v)

        def idx_slice(g3, gg):
            q = g3 // nh
            h = lax.rem(g3, nh)
            return idx_v.at[pl.ds(q * (G * S) + gg * S + h * QL, QL)]

        def start_gathers(g3, b):
            for gg in range(G):
                pltpu.async_copy(
                    w_hbm.at[idx_slice(g3, gg)],
                    rows_v.at[b, gg],
                    gsems[b],
                )

        def wait_gathers(g3, b):
            for gg in range(G):
                pltpu.make_async_copy(
                    w_hbm.at[idx_slice(g3, gg)],
                    rows_v.at[b, gg],
                    gsems[b],
                ).wait()

        def out_slice(g3):
            q = g3 // nh
            h = lax.rem(g3, nh)
            seq0 = wid * seq_per_tile + q * G
            return out_hbm.at[pl.ds(seq0, G), pl.ds(h * QL, QL)]

        def start_stores(g3, b):
            pltpu.async_copy(rows_v.at[b], out_slice(g3), ssems[b])

        def wait_stores(g3, b):
            pltpu.make_async_copy(rows_v.at[b], out_slice(g3), ssems[b]).wait()

        start_gathers(0, 0)
        pltpu.sync_copy(pe_hbm.at[pl.ds(0, S)], pe_v)

        def group_body(g3, b):
            nb = 1 - b

            @pl.when(g3 >= 1)
            def _():
                wait_stores(g3 - 1, nb)

            @pl.when(g3 + 1 < NGRP)
            def _():
                start_gathers(g3 + 1, nb)

            wait_gathers(g3, b)

            p0 = lax.rem(g3, nh) * QL

            @plsc.parallel_loop(0, QL, unroll=2)
            def _(i):
                for j in range(nj):
                    sl = pl.ds(j * _LANES, _LANES)
                    p = pe_v[p0 + i, sl]
                    for gg in range(G):
                        r = rows_v[b, gg, i, sl]
                        rows_v[b, gg, i, sl] = r * scale + p

            start_stores(g3, b)

        def outer(t, carry):
            for b in range(2):
                group_body(t * 2 + b, b)
            return carry

        lax.fori_loop(0, NGRP // 2, outer, 0)

        wait_stores(NGRP - 1, 1)

    return k


def kernel(input_seq, W, pos_emb):
    B, S = input_seq.shape
    _, D = W.shape
    N = B * S

    info = plsc.get_sparse_core_info()
    NC, NS = info.num_cores, info.num_subcores
    NW = NC * NS

    G = 8
    QL = 40
    assert D % _LANES == 0
    assert S % QL == 0 and QL % 8 == 0
    assert B % (NW * G) == 0
    NGRP = (B // NW // G) * (S // QL)
    assert NGRP % 2 == 0

    k = _make_sc_kernel(N, D, NW, NC, S, G, QL, NGRP)
    idx = input_seq.astype(jnp.int32).reshape(-1)
    return k(W, idx, pos_emb.astype(jnp.float32))
